# Initial kernel scaffold; baseline (speedup 1.0000x reference)
import jax
import jax.numpy as jnp
from jax.experimental import pallas as pl


def kernel(x, cond, t, edge_idx, edge_dist, params):
    raise NotImplementedError("write your pallas kernel here")


# speedup vs baseline: 4.0251x; 4.0251x over previous
import functools

import jax
import jax.numpy as jnp
from jax import lax
from jax.experimental import pallas as pl
from jax.experimental.pallas import tpu as pltpu
from jax.experimental.pallas import tpu_sc as plsc

NC, NS = 2, 16
NW = NC * NS
CH = 80
HD = 128
BE = 1600
NACC = 640 * NS


def _mesh():
    return plsc.VectorSubcoreMesh(core_axis_name="c", subcore_axis_name="s")


_SC_PARAMS = pltpu.CompilerParams(needs_layout_passes=False)


def _rsqrt16(d2):
    i = plsc.bitcast(d2, jnp.int32)
    y = plsc.bitcast(jnp.int32(0x5F3759DF) - (i >> 1), jnp.float32)
    for _ in range(3):
        y = y * (1.5 - 0.5 * d2 * y * y)
    return y


def _gather_body(nchunks, t_hbm, src_hbm, dst_hbm, gs_hbm, gd_hbm,
                 idx_s, idx_d, rows_s, rows_d, sem_s, sem_d):
    cid = lax.axis_index("c")
    sid = lax.axis_index("s")
    wid = sid * NC + cid
    ebase = wid * nchunks * CH
    pltpu.sync_copy(src_hbm.at[wid], idx_s)
    pltpu.sync_copy(dst_hbm.at[wid], idx_d)

    def body(j, carry):
        cs = pltpu.async_copy(t_hbm.at[idx_s.at[j]], rows_s, sem_s)
        cd = pltpu.async_copy(t_hbm.at[idx_d.at[j]], rows_d, sem_d)
        cs.wait()
        pltpu.sync_copy(rows_s, gs_hbm.at[pl.ds(ebase + j * CH, CH)])
        cd.wait()
        pltpu.sync_copy(rows_d, gd_hbm.at[pl.ds(ebase + j * CH, CH)])
        return carry

    lax.fori_loop(0, nchunks, body, 0)


def _sc_gather(t_tab, src3d, dst3d):
    nchunks = src3d.shape[1]
    e = NW * nchunks * CH
    kfn = pl.kernel(
        functools.partial(_gather_body, nchunks),
        out_type=(jax.ShapeDtypeStruct((e, HD), jnp.float32),
                  jax.ShapeDtypeStruct((e, HD), jnp.float32)),
        mesh=_mesh(),
        compiler_params=_SC_PARAMS,
        scratch_types=[
            pltpu.VMEM((nchunks, CH), jnp.int32),
            pltpu.VMEM((nchunks, CH), jnp.int32),
            pltpu.VMEM((CH, HD), jnp.float32),
            pltpu.VMEM((CH, HD), jnp.float32),
            pltpu.SemaphoreType.DMA,
            pltpu.SemaphoreType.DMA,
        ],
    )
    return kfn(t_tab, src3d, dst3d)


def _x_chunk(idx_s, idx_d, cwv, xt, accx, j):
    for g in range(CH // 16):
        s16 = idx_s[j, pl.ds(g * 16, 16)]
        d16 = idx_d[j, pl.ds(g * 16, 16)]
        xs = [plsc.load_gather(xt[c], [s16]) for c in range(3)]
        xd = [plsc.load_gather(xt[c], [d16]) for c in range(3)]
        dx = [a - b for a, b in zip(xs, xd)]
        d2 = dx[0] * dx[0] + dx[1] * dx[1] + dx[2] * dx[2]
        r = jnp.minimum(_rsqrt16(d2), 1e8)
        w = cwv[pl.ds(g * 16, 16)] * r
        for c in range(3):
            plsc.addupdate_scatter(accx[c], [d16], dx[c] * w)


def _scatter_h_body(nchunks, m_hbm, dst_hbm, zh_hbm, ph_hbm,
                    idx_d, vals, acc):
    cid = lax.axis_index("c")
    sid = lax.axis_index("s")
    wid = sid * NC + cid
    ebase = wid * nchunks * CH
    rpt = NACC // NS
    r0 = sid * rpt
    pltpu.sync_copy(zh_hbm.at[pl.ds(r0, rpt)], acc.at[pl.ds(r0, rpt)])
    pltpu.sync_copy(dst_hbm.at[wid], idx_d)
    plsc.subcore_barrier()

    def body(j, carry):
        pltpu.sync_copy(m_hbm.at[pl.ds(ebase + j * CH, CH)], vals)
        pltpu.sync_copy(vals, acc.at[idx_d.at[j]], add=True)
        return carry

    lax.fori_loop(0, nchunks, body, 0)
    plsc.subcore_barrier()
    pltpu.sync_copy(acc.at[pl.ds(r0, rpt)], ph_hbm.at[cid, pl.ds(r0, rpt)])


def _scatter_x_body(nchunks, cw_hbm, src_hbm, dst_hbm,
                    xt0_hbm, xt1_hbm, xt2_hbm, z1_hbm,
                    px0_hbm, px1_hbm, px2_hbm,
                    idx_s, idx_d, cwv, xtb0, xtb1, xtb2, ax0, ax1, ax2):
    cid = lax.axis_index("c")
    sid = lax.axis_index("s")
    wid = sid * NC + cid
    ebase = wid * nchunks * CH
    pltpu.sync_copy(src_hbm.at[wid], idx_s)
    pltpu.sync_copy(dst_hbm.at[wid], idx_d)
    xt = (xtb0, xtb1, xtb2)
    accx = (ax0, ax1, ax2)
    for dstp, srcp in zip(xt, (xt0_hbm, xt1_hbm, xt2_hbm)):
        pltpu.sync_copy(srcp, dstp)
    for a in accx:
        pltpu.sync_copy(z1_hbm, a)

    def body(j, carry):
        pltpu.sync_copy(cw_hbm.at[pl.ds(ebase + j * CH, CH)], cwv)
        _x_chunk(idx_s, idx_d, cwv, xt, accx, j)
        return carry

    lax.fori_loop(0, nchunks, body, 0)
    for a, out in zip(accx, (px0_hbm, px1_hbm, px2_hbm)):
        pltpu.sync_copy(a, out.at[wid])


def _sc_scatter_h(m_rows, dst3d, zh):
    nchunks = dst3d.shape[1]
    kfn = pl.kernel(
        functools.partial(_scatter_h_body, nchunks),
        out_type=jax.ShapeDtypeStruct((NC, NACC, HD), jnp.float32),
        mesh=_mesh(),
        compiler_params=_SC_PARAMS,
        scratch_types=[
            pltpu.VMEM((nchunks, CH), jnp.int32),
            pltpu.VMEM((CH, HD), jnp.float32),
            pltpu.VMEM_SHARED((NACC, HD), jnp.float32),
        ],
    )
    return kfn(m_rows, dst3d, zh)


def _sc_scatter_x(cw, src3d, dst3d, xcols, z1):
    nn = z1.shape[0]
    nchunks = dst3d.shape[1]
    kfn = pl.kernel(
        functools.partial(_scatter_x_body, nchunks),
        out_type=(jax.ShapeDtypeStruct((NW, nn), jnp.float32),
                  jax.ShapeDtypeStruct((NW, nn), jnp.float32),
                  jax.ShapeDtypeStruct((NW, nn), jnp.float32)),
        mesh=_mesh(),
        compiler_params=_SC_PARAMS,
        scratch_types=[
            pltpu.VMEM((nchunks, CH), jnp.int32),
            pltpu.VMEM((nchunks, CH), jnp.int32),
            pltpu.VMEM((CH,), jnp.float32),
            pltpu.VMEM((nn,), jnp.float32),
            pltpu.VMEM((nn,), jnp.float32),
            pltpu.VMEM((nn,), jnp.float32),
            pltpu.VMEM((nn,), jnp.float32),
            pltpu.VMEM((nn,), jnp.float32),
            pltpu.VMEM((nn,), jnp.float32),
        ],
    )
    return kfn(cw, src3d, dst3d, *xcols, z1)


def _edge_attr(d, ew1, eb1, ew2, eb2):
    ea = d * ew1[...] + eb1[...][None, :]
    return jax.nn.silu(ea) @ ew2[...] + eb2[...][None, :]


def _mlp_body(gs, gd, dist, w1, b1, w2, b2, cw2t, ew1, eb1, ew2, eb2,
              m_out, cw_out):
    ea = _edge_attr(dist[...], ew1, eb1, ew2, eb2)
    minp = jnp.concatenate([gs[...], gd[...], ea], axis=1)
    a = minp @ w1[...] + b1[...][None, :]
    an = jax.nn.silu(a[:, :256])
    ac = jax.nn.silu(a[:, 256:])
    m_out[...] = an @ w2[...] + b2[...][None, :]
    cw_out[...] = jnp.sum(ac * cw2t[...], axis=1, keepdims=True)


def _mlp_c_body(gs, gd, dist, w1, b1, cw2t, ew1, eb1, ew2, eb2, cw_out):
    ea = _edge_attr(dist[...], ew1, eb1, ew2, eb2)
    minp = jnp.concatenate([gs[...], gd[...], ea], axis=1)
    ac = jax.nn.silu(minp @ w1[...] + b1[...][None, :])
    cw_out[...] = jnp.sum(ac * cw2t[...], axis=1, keepdims=True)


def _full(shape):
    return pl.BlockSpec(shape, lambda i: (0,) * len(shape))


def _tc_mlp(gs, gd, dist, p):
    e = gs.shape[0]
    w1 = jnp.concatenate([p['n_W1'], p['c_W1']], axis=1)
    b1 = jnp.concatenate([p['n_b1'], p['c_b1']], axis=0)
    cw2t = p['c_W2'].reshape(1, 256)
    m, cw = pl.pallas_call(
        _mlp_body,
        grid=(e // BE,),
        in_specs=[
            pl.BlockSpec((BE, HD), lambda i: (i, 0)),
            pl.BlockSpec((BE, HD), lambda i: (i, 0)),
            pl.BlockSpec((BE, 1), lambda i: (i, 0)),
            _full((272, 512)), _full((512,)),
            _full((256, HD)), _full((HD,)),
            _full((1, 256)),
            _full((1, 16)), _full((16,)), _full((16, 16)), _full((16,)),
        ],
        out_specs=(pl.BlockSpec((BE, HD), lambda i: (i, 0)),
                   pl.BlockSpec((BE, 1), lambda i: (i, 0))),
        out_shape=(jax.ShapeDtypeStruct((e, HD), jnp.float32),
                   jax.ShapeDtypeStruct((e, 1), jnp.float32)),
    )(gs, gd, dist, w1, b1, p['n_W2'], p['n_b2'], cw2t,
      p['e_W1'], p['e_b1'], p['e_W2'], p['e_b2'])
    return m, cw.reshape(e)


def _tc_mlp_c(gs, gd, dist, p):
    e = gs.shape[0]
    cw2t = p['c_W2'].reshape(1, 256)
    cw = pl.pallas_call(
        _mlp_c_body,
        grid=(e // BE,),
        in_specs=[
            pl.BlockSpec((BE, HD), lambda i: (i, 0)),
            pl.BlockSpec((BE, HD), lambda i: (i, 0)),
            pl.BlockSpec((BE, 1), lambda i: (i, 0)),
            _full((272, 256)), _full((256,)),
            _full((1, 256)),
            _full((1, 16)), _full((16,)), _full((16, 16)), _full((16,)),
        ],
        out_specs=pl.BlockSpec((BE, 1), lambda i: (i, 0)),
        out_shape=jax.ShapeDtypeStruct((e, 1), jnp.float32),
    )(gs, gd, dist, p['c_W1'], p['c_b1'], cw2t,
      p['e_W1'], p['e_b1'], p['e_W2'], p['e_b2'])
    return cw.reshape(e)


def _update_h_body(t, p0, p1, out):
    out[...] = t[...] + p0[...] + p1[...]


def _tc_update_h(t_tab, parts):
    nn = t_tab.shape[0]
    blk = nn // 10
    return pl.pallas_call(
        _update_h_body,
        grid=(nn // blk,),
        in_specs=[pl.BlockSpec((blk, HD), lambda i: (i, 0))] * 3,
        out_specs=pl.BlockSpec((blk, HD), lambda i: (i, 0)),
        out_shape=jax.ShapeDtypeStruct((nn, HD), jnp.float32),
    )(t_tab, parts[0], parts[1])


def _update_x_body(x0, x1, x2, p0, p1, p2, o0, o1, o2):
    o0[...] = x0[...] + jnp.sum(p0[...], axis=0, keepdims=True)
    o1[...] = x1[...] + jnp.sum(p1[...], axis=0, keepdims=True)
    o2[...] = x2[...] + jnp.sum(p2[...], axis=0, keepdims=True)


def _tc_update_x(xcols, pxs):
    nn = xcols[0].shape[0]
    x2d = [c.reshape(1, nn) for c in xcols]
    out = pl.pallas_call(
        _update_x_body,
        grid=(1,),
        in_specs=[_full((1, nn))] * 3 + [_full((NW, nn))] * 3,
        out_specs=(_full((1, nn)),) * 3,
        out_shape=(jax.ShapeDtypeStruct((1, nn), jnp.float32),) * 3,
    )(*x2d, *pxs)
    return [o.reshape(nn) for o in out]


def kernel(x, cond, t, edge_idx, edge_dist, params):
    b, n, _ = x.shape
    nn = b * n
    e = edge_dist.shape[0]
    src3d = edge_idx[0].astype(jnp.int32).reshape(NW, e // (NW * CH), CH)
    dst3d = edge_idx[1].astype(jnp.int32).reshape(NW, e // (NW * CH), CH)
    zh = jnp.zeros((NACC, HD), jnp.float32)
    z1 = jnp.zeros((nn,), jnp.float32)

    t_col = jnp.full((nn, 1), jnp.asarray(t, jnp.float32))
    t_tab = jnp.concatenate(
        [cond.reshape(nn, -1).astype(jnp.float32), t_col], axis=1)
    xf = x.reshape(nn, 3).astype(jnp.float32)
    xcols = [xf[:, c] for c in range(3)]
    dist = edge_dist.astype(jnp.float32).reshape(e, 1)

    for li, p in enumerate(params):
        gs, gd = _sc_gather(t_tab, src3d, dst3d)
        last = li == len(params) - 1
        if last:
            cw = _tc_mlp_c(gs, gd, dist, p)
            pxs = _sc_scatter_x(cw, src3d, dst3d, xcols, z1)
        else:
            m_rows, cw = _tc_mlp(gs, gd, dist, p)
            ph = _sc_scatter_h(m_rows, dst3d, zh)
            pxs = _sc_scatter_x(cw, src3d, dst3d, xcols, z1)
            t_tab = _tc_update_h(t_tab, ph[:, :nn])
        xcols = _tc_update_x(xcols, pxs)

    return jnp.stack(xcols, axis=1).reshape(b, n, 3)


# speedup vs baseline: 4.5146x; 1.1216x over previous
import functools

import jax
import jax.numpy as jnp
from jax import lax
from jax.experimental import pallas as pl
from jax.experimental.pallas import tpu as pltpu
from jax.experimental.pallas import tpu_sc as plsc

NC, NS = 2, 16
NW = NC * NS
CH = 80
HD = 128
BE = 1600
NACC = 632 * NS
RG = 4
RS = 4


def _mesh():
    return plsc.VectorSubcoreMesh(core_axis_name="c", subcore_axis_name="s")


_SC_PARAMS = pltpu.CompilerParams(needs_layout_passes=False)


def _rsqrt16(d2):
    i = plsc.bitcast(d2, jnp.int32)
    y = plsc.bitcast(jnp.int32(0x5F3759DF) - (i >> 1), jnp.float32)
    for _ in range(3):
        y = y * (1.5 - 0.5 * d2 * y * y)
    return y


def _gather_body(nchunks, t_hbm, src_hbm, dst_hbm, gs_hbm, gd_hbm,
                 idx_s, idx_d, rows_s, rows_d, sems, semd):
    cid = lax.axis_index("c")
    sid = lax.axis_index("s")
    wid = sid * NC + cid
    ebase = wid * nchunks * CH
    pltpu.sync_copy(src_hbm.at[wid], idx_s)
    pltpu.sync_copy(dst_hbm.at[wid], idx_d)

    def start(b, j):
        pltpu.async_copy(t_hbm.at[idx_s.at[j]], rows_s.at[b], sems.at[b])
        pltpu.async_copy(t_hbm.at[idx_d.at[j]], rows_d.at[b], semd.at[b])

    def drain(b, j):
        pltpu.make_async_copy(t_hbm.at[idx_s.at[j]], rows_s.at[b],
                              sems.at[b]).wait()
        pltpu.sync_copy(rows_s.at[b], gs_hbm.at[pl.ds(ebase + j * CH, CH)])
        pltpu.make_async_copy(t_hbm.at[idx_d.at[j]], rows_d.at[b],
                              semd.at[b]).wait()
        pltpu.sync_copy(rows_d.at[b], gd_hbm.at[pl.ds(ebase + j * CH, CH)])

    for b in range(RG):
        start(b, b)

    def body(g, carry):
        for b in range(RG):
            j = g * RG + b
            drain(b, j)
            start(b, j + RG)
        return carry

    nfull = nchunks // RG - 1
    lax.fori_loop(0, nfull, body, 0)
    base = nfull * RG
    for b in range(RG):
        drain(b, base + b)
    for r in range(nchunks - base - RG):
        start(r % RG, base + RG + r)
        drain(r % RG, base + RG + r)


def _sc_gather(t_tab, src3d, dst3d):
    nchunks = src3d.shape[1]
    e = NW * nchunks * CH
    kfn = pl.kernel(
        functools.partial(_gather_body, nchunks),
        out_type=(jax.ShapeDtypeStruct((e, HD), jnp.float32),
                  jax.ShapeDtypeStruct((e, HD), jnp.float32)),
        mesh=_mesh(),
        compiler_params=_SC_PARAMS,
        scratch_types=[
            pltpu.VMEM((nchunks, CH), jnp.int32),
            pltpu.VMEM((nchunks, CH), jnp.int32),
            pltpu.VMEM((RG, CH, HD), jnp.float32),
            pltpu.VMEM((RG, CH, HD), jnp.float32),
            pltpu.SemaphoreType.DMA((RG,)),
            pltpu.SemaphoreType.DMA((RG,)),
        ],
    )
    return kfn(t_tab, src3d, dst3d)


def _x_chunk(idx_s, idx_d, cwv, xt, accx, j):
    for g in range(CH // 16):
        s16 = idx_s[j, pl.ds(g * 16, 16)]
        d16 = idx_d[j, pl.ds(g * 16, 16)]
        xs = [plsc.load_gather(xt[c], [s16]) for c in range(3)]
        xd = [plsc.load_gather(xt[c], [d16]) for c in range(3)]
        dx = [a - b for a, b in zip(xs, xd)]
        d2 = dx[0] * dx[0] + dx[1] * dx[1] + dx[2] * dx[2]
        r = jnp.minimum(_rsqrt16(d2), 1e8)
        w = cwv[pl.ds(g * 16, 16)] * r
        for c in range(3):
            plsc.addupdate_scatter(accx[c], [d16], dx[c] * w)


def _scatter_h_body(nchunks, m_hbm, dst_hbm, zh_hbm, ph_hbm,
                    idxr, vals, acc, semv, semi):
    cid = lax.axis_index("c")
    sid = lax.axis_index("s")
    wid = sid * NC + cid
    ebase = wid * nchunks * CH
    rpt = NACC // NS
    r0 = sid * rpt
    pltpu.sync_copy(zh_hbm.at[pl.ds(r0, rpt)], acc.at[pl.ds(r0, rpt)])
    plsc.subcore_barrier()

    def start(b, j):
        pltpu.async_copy(m_hbm.at[pl.ds(ebase + j * CH, CH)], vals.at[b],
                         semv.at[b])
        pltpu.async_copy(dst_hbm.at[pl.ds(ebase + j * CH, CH)], idxr.at[b],
                         semi.at[b])

    def drain(b, j):
        pltpu.make_async_copy(m_hbm.at[pl.ds(ebase + j * CH, CH)],
                              vals.at[b], semv.at[b]).wait()
        pltpu.make_async_copy(dst_hbm.at[pl.ds(ebase + j * CH, CH)],
                              idxr.at[b], semi.at[b]).wait()
        pltpu.sync_copy(vals.at[b], acc.at[idxr.at[b]], add=True)

    for b in range(RS):
        start(b, b)

    def body(g, carry):
        for b in range(RS):
            j = g * RS + b
            drain(b, j)
            start(b, j + RS)
        return carry

    nfull = nchunks // RS - 1
    lax.fori_loop(0, nfull, body, 0)
    base = nfull * RS
    for b in range(RS):
        drain(b, base + b)
    for r in range(nchunks - base - RS):
        start(r % RS, base + RS + r)
        drain(r % RS, base + RS + r)

    plsc.subcore_barrier()
    pltpu.sync_copy(acc.at[pl.ds(r0, rpt)], ph_hbm.at[cid, pl.ds(r0, rpt)])


def _scatter_x_body(nchunks, cw_hbm, src_hbm, dst_hbm,
                    xt0_hbm, xt1_hbm, xt2_hbm, z1_hbm,
                    px0_hbm, px1_hbm, px2_hbm,
                    idx_s, idx_d, cwv, xtb0, xtb1, xtb2, ax0, ax1, ax2):
    cid = lax.axis_index("c")
    sid = lax.axis_index("s")
    wid = sid * NC + cid
    ebase = wid * nchunks * CH
    pltpu.sync_copy(src_hbm.at[wid], idx_s)
    pltpu.sync_copy(dst_hbm.at[wid], idx_d)
    xt = (xtb0, xtb1, xtb2)
    accx = (ax0, ax1, ax2)
    for dstp, srcp in zip(xt, (xt0_hbm, xt1_hbm, xt2_hbm)):
        pltpu.sync_copy(srcp, dstp)
    for a in accx:
        pltpu.sync_copy(z1_hbm, a)

    def body(j, carry):
        pltpu.sync_copy(cw_hbm.at[pl.ds(ebase + j * CH, CH)], cwv)
        _x_chunk(idx_s, idx_d, cwv, xt, accx, j)
        return carry

    lax.fori_loop(0, nchunks, body, 0)
    for a, out in zip(accx, (px0_hbm, px1_hbm, px2_hbm)):
        pltpu.sync_copy(a, out.at[wid])


def _sc_scatter_h(m_rows, dst1d, zh):
    nchunks = dst1d.shape[0] // (NW * CH)
    kfn = pl.kernel(
        functools.partial(_scatter_h_body, nchunks),
        out_type=jax.ShapeDtypeStruct((NC, NACC, HD), jnp.float32),
        mesh=_mesh(),
        compiler_params=_SC_PARAMS,
        scratch_types=[
            pltpu.VMEM((RS, CH), jnp.int32),
            pltpu.VMEM((RS, CH, HD), jnp.float32),
            pltpu.VMEM_SHARED((NACC, HD), jnp.float32),
            pltpu.SemaphoreType.DMA((RS,)),
            pltpu.SemaphoreType.DMA((RS,)),
        ],
    )
    return kfn(m_rows, dst1d, zh)


def _sc_scatter_x(cw, src3d, dst3d, xcols, z1):
    nn = z1.shape[0]
    nchunks = dst3d.shape[1]
    kfn = pl.kernel(
        functools.partial(_scatter_x_body, nchunks),
        out_type=(jax.ShapeDtypeStruct((NW, nn), jnp.float32),
                  jax.ShapeDtypeStruct((NW, nn), jnp.float32),
                  jax.ShapeDtypeStruct((NW, nn), jnp.float32)),
        mesh=_mesh(),
        compiler_params=_SC_PARAMS,
        scratch_types=[
            pltpu.VMEM((nchunks, CH), jnp.int32),
            pltpu.VMEM((nchunks, CH), jnp.int32),
            pltpu.VMEM((CH,), jnp.float32),
            pltpu.VMEM((nn,), jnp.float32),
            pltpu.VMEM((nn,), jnp.float32),
            pltpu.VMEM((nn,), jnp.float32),
            pltpu.VMEM((nn,), jnp.float32),
            pltpu.VMEM((nn,), jnp.float32),
            pltpu.VMEM((nn,), jnp.float32),
        ],
    )
    return kfn(cw, src3d, dst3d, *xcols, z1)


def _edge_attr(d, ew1, eb1, ew2, eb2):
    ea = d * ew1[...] + eb1[...][None, :]
    return jax.nn.silu(ea) @ ew2[...] + eb2[...][None, :]


def _mlp_body(gs, gd, dist, w1, b1, w2, b2, cw2t, ew1, eb1, ew2, eb2,
              m_out, cw_out):
    ea = _edge_attr(dist[...], ew1, eb1, ew2, eb2)
    minp = jnp.concatenate([gs[...], gd[...], ea], axis=1)
    a = minp @ w1[...] + b1[...][None, :]
    an = jax.nn.silu(a[:, :256])
    ac = jax.nn.silu(a[:, 256:])
    m_out[...] = an @ w2[...] + b2[...][None, :]
    cw_out[...] = jnp.sum(ac * cw2t[...], axis=1, keepdims=True)


def _mlp_c_body(gs, gd, dist, w1, b1, cw2t, ew1, eb1, ew2, eb2, cw_out):
    ea = _edge_attr(dist[...], ew1, eb1, ew2, eb2)
    minp = jnp.concatenate([gs[...], gd[...], ea], axis=1)
    ac = jax.nn.silu(minp @ w1[...] + b1[...][None, :])
    cw_out[...] = jnp.sum(ac * cw2t[...], axis=1, keepdims=True)


def _full(shape):
    return pl.BlockSpec(shape, lambda i: (0,) * len(shape))


def _tc_mlp(gs, gd, dist, p):
    e = gs.shape[0]
    w1 = jnp.concatenate([p['n_W1'], p['c_W1']], axis=1)
    b1 = jnp.concatenate([p['n_b1'], p['c_b1']], axis=0)
    cw2t = p['c_W2'].reshape(1, 256)
    m, cw = pl.pallas_call(
        _mlp_body,
        grid=(e // BE,),
        in_specs=[
            pl.BlockSpec((BE, HD), lambda i: (i, 0)),
            pl.BlockSpec((BE, HD), lambda i: (i, 0)),
            pl.BlockSpec((BE, 1), lambda i: (i, 0)),
            _full((272, 512)), _full((512,)),
            _full((256, HD)), _full((HD,)),
            _full((1, 256)),
            _full((1, 16)), _full((16,)), _full((16, 16)), _full((16,)),
        ],
        out_specs=(pl.BlockSpec((BE, HD), lambda i: (i, 0)),
                   pl.BlockSpec((BE, 1), lambda i: (i, 0))),
        out_shape=(jax.ShapeDtypeStruct((e, HD), jnp.float32),
                   jax.ShapeDtypeStruct((e, 1), jnp.float32)),
    )(gs, gd, dist, w1, b1, p['n_W2'], p['n_b2'], cw2t,
      p['e_W1'], p['e_b1'], p['e_W2'], p['e_b2'])
    return m, cw.reshape(e)


def _tc_mlp_c(gs, gd, dist, p):
    e = gs.shape[0]
    cw2t = p['c_W2'].reshape(1, 256)
    cw = pl.pallas_call(
        _mlp_c_body,
        grid=(e // BE,),
        in_specs=[
            pl.BlockSpec((BE, HD), lambda i: (i, 0)),
            pl.BlockSpec((BE, HD), lambda i: (i, 0)),
            pl.BlockSpec((BE, 1), lambda i: (i, 0)),
            _full((272, 256)), _full((256,)),
            _full((1, 256)),
            _full((1, 16)), _full((16,)), _full((16, 16)), _full((16,)),
        ],
        out_specs=pl.BlockSpec((BE, 1), lambda i: (i, 0)),
        out_shape=jax.ShapeDtypeStruct((e, 1), jnp.float32),
    )(gs, gd, dist, p['c_W1'], p['c_b1'], cw2t,
      p['e_W1'], p['e_b1'], p['e_W2'], p['e_b2'])
    return cw.reshape(e)


def _update_h_body(t, p0, p1, out):
    out[...] = t[...] + p0[...] + p1[...]


def _tc_update_h(t_tab, parts):
    nn = t_tab.shape[0]
    blk = nn // 10
    return pl.pallas_call(
        _update_h_body,
        grid=(nn // blk,),
        in_specs=[pl.BlockSpec((blk, HD), lambda i: (i, 0))] * 3,
        out_specs=pl.BlockSpec((blk, HD), lambda i: (i, 0)),
        out_shape=jax.ShapeDtypeStruct((nn, HD), jnp.float32),
    )(t_tab, parts[0], parts[1])


def _update_x_body(x0, x1, x2, p0, p1, p2, o0, o1, o2):
    o0[...] = x0[...] + jnp.sum(p0[...], axis=0, keepdims=True)
    o1[...] = x1[...] + jnp.sum(p1[...], axis=0, keepdims=True)
    o2[...] = x2[...] + jnp.sum(p2[...], axis=0, keepdims=True)


def _tc_update_x(xcols, pxs):
    nn = xcols[0].shape[0]
    x2d = [c.reshape(1, nn) for c in xcols]
    out = pl.pallas_call(
        _update_x_body,
        grid=(1,),
        in_specs=[_full((1, nn))] * 3 + [_full((NW, nn))] * 3,
        out_specs=(_full((1, nn)),) * 3,
        out_shape=(jax.ShapeDtypeStruct((1, nn), jnp.float32),) * 3,
    )(*x2d, *pxs)
    return [o.reshape(nn) for o in out]


def kernel(x, cond, t, edge_idx, edge_dist, params):
    b, n, _ = x.shape
    nn = b * n
    e = edge_dist.shape[0]
    src3d = edge_idx[0].astype(jnp.int32).reshape(NW, e // (NW * CH), CH)
    dst3d = edge_idx[1].astype(jnp.int32).reshape(NW, e // (NW * CH), CH)
    dst1d = edge_idx[1].astype(jnp.int32)
    zh = jnp.zeros((NACC, HD), jnp.float32)
    z1 = jnp.zeros((nn,), jnp.float32)

    t_col = jnp.full((nn, 1), jnp.asarray(t, jnp.float32))
    t_tab = jnp.concatenate(
        [cond.reshape(nn, -1).astype(jnp.float32), t_col], axis=1)
    xf = x.reshape(nn, 3).astype(jnp.float32)
    xcols = [xf[:, c] for c in range(3)]
    dist = edge_dist.astype(jnp.float32).reshape(e, 1)

    for li, p in enumerate(params):
        gs, gd = _sc_gather(t_tab, src3d, dst3d)
        last = li == len(params) - 1
        if last:
            cw = _tc_mlp_c(gs, gd, dist, p)
            pxs = _sc_scatter_x(cw, src3d, dst3d, xcols, z1)
        else:
            m_rows, cw = _tc_mlp(gs, gd, dist, p)
            ph = _sc_scatter_h(m_rows, dst1d, zh)
            pxs = _sc_scatter_x(cw, src3d, dst3d, xcols, z1)
            t_tab = _tc_update_h(t_tab, ph[:, :nn])
        xcols = _tc_update_x(xcols, pxs)

    return jnp.stack(xcols, axis=1).reshape(b, n, 3)
